# baseline (speedup 1.0000x reference)
import functools

import jax
import jax.numpy as jnp
import numpy as np
from jax import lax
from jax.experimental import pallas as pl
from jax.experimental.pallas import tpu as pltpu
from jax.experimental.pallas import tpu_sc as plsc

LABEL_SMOOTHING = 0.1
VOCAB = 100000
CONFIDENCE = 1.0 - LABEL_SMOOTHING
BATCH = 1024
IGN = VOCAB - 100

NC, NS, L = 2, 16, 16
NW = NC * NS
B_PER_W = BATCH // NW
IGN_C0 = (IGN // L) * L
IGN_LANE = IGN - IGN_C0


TILE_R, TILE_C = 8, 128
TAIL0 = (VOCAB // TILE_C) * TILE_C
TAIL_W = VOCAB - TAIL0
LAST_FULL = TAIL0 - TILE_C
IGN_T0 = (IGN // TILE_C) * TILE_C
N_RB = B_PER_W // TILE_R


R_SC = 1024
ROWS_W = R_SC // NW
N_RB_S = ROWS_W // TILE_R
PANEL = 512
HALF = 8 * PANEL
N_HALF = 24
TAIL_S0 = N_HALF * HALF
TAIL_SW = VOCAB - TAIL_S0


def _sum_half(sbuf_v, s0, accs):
    for b in range(8):
        def body(i, a, _b=b):
            c = i * (4 * L)
            a = list(a)
            for cc in range(4):
                for r in range(8):
                    a[r] = a[r] + sbuf_v[s0 + _b, r, pl.ds(c + cc * L, L)]
            return tuple(a)
        accs = lax.fori_loop(0, PANEL // (4 * L), body, accs)
    return accs


def _fire_half(out_hbm, sbuf_v, r0, col0, s0, sem):
    for b in range(8):
        pltpu.async_copy(
            out_hbm.at[pl.ds(r0, TILE_R),
                       pl.ds(pl.multiple_of(col0 + b * PANEL, 128), PANEL)],
            sbuf_v.at[s0 + b], sem)


def _drain_half(out_hbm, sbuf_v, s0, sem):
    for b in range(8):
        pltpu.make_async_copy(
            out_hbm.at[pl.ds(0, TILE_R), pl.ds(0, PANEL)],
            sbuf_v.at[s0 + b], sem).wait()


def _sc_gather_body(out_hbm, tgt_hbm, part_hbm, tgt_v, gbuf_v, tbuf_v,
                    ibuf_v, sbuf_v, stail_v, st_v, sem, sem_a, sem_b):
    wid = lax.axis_index("s") * NC + lax.axis_index("c")
    base = wid * B_PER_W
    pltpu.sync_copy(tgt_hbm.at[pl.ds(base, B_PER_W)], tgt_v)
    tvecs = [tgt_v[pl.ds(k * L, L)] for k in range(B_PER_W // L)]
    ts = [tvecs[i // L][i % L] for i in range(B_PER_W)]
    copies = []
    for k in range(N_RB):
        r0 = base + k * TILE_R
        copies.append(pltpu.async_copy(
            out_hbm.at[pl.ds(r0, TILE_R), pl.ds(TAIL0, TAIL_W)],
            tbuf_v.at[k], sem))
        copies.append(pltpu.async_copy(
            out_hbm.at[pl.ds(r0, TILE_R), pl.ds(IGN_T0, TILE_C)],
            ibuf_v.at[k], sem))
    for i in range(B_PER_W):
        t = ts[i]
        c0 = pl.multiple_of(
            jnp.minimum((t >> 7) << 7, LAST_FULL), TILE_C)
        r0 = base + (i // TILE_R) * TILE_R
        copies.append(pltpu.async_copy(
            out_hbm.at[pl.ds(r0, TILE_R), pl.ds(c0, TILE_C)],
            gbuf_v.at[i], sem))
    for c in copies:
        c.wait()
    lanes = lax.iota(jnp.int32, L)
    acc_g = jnp.zeros((L,), jnp.float32)
    acc_h = jnp.zeros((L,), jnp.float32)
    acc_c = jnp.zeros((L,), jnp.float32)
    for i in range(B_PER_W):
        t = ts[i]
        below_f = jnp.where(t < TAIL0, 1.0, 0.0)
        is_ign_f = jnp.where(t == IGN, 1.0, 0.0)
        off = t - jnp.minimum((t >> 7) << 7, LAST_FULL)
        sub = jnp.minimum((off >> 4) << 4, TILE_C - L)
        vm = gbuf_v[i, i % TILE_R, pl.ds(sub, L)]
        lane_m = jnp.full((L,), off - sub, jnp.int32)
        acc_g = acc_g + jnp.where(lanes == lane_m, vm, 0.0) * jnp.full(
            (L,), below_f, jnp.float32)
        offt = jnp.maximum(t - TAIL0, 0)
        subt = jnp.minimum((offt >> 4) << 4, TAIL_W - L)
        vt = tbuf_v[i // TILE_R, i % TILE_R, pl.ds(subt, L)]
        lane_t = jnp.full((L,), offt - subt, jnp.int32)
        acc_g = acc_g + jnp.where(lanes == lane_t, vt, 0.0) * jnp.full(
            (L,), 1.0 - below_f, jnp.float32)
        ign_sub = ((IGN - IGN_T0) // L) * L
        ign_lane = (IGN - IGN_T0) - ign_sub
        vi = ibuf_v[i // TILE_R, i % TILE_R, pl.ds(ign_sub, L)]
        acc_h = acc_h + jnp.where(lanes == ign_lane, vi, 0.0) * jnp.full(
            (L,), 1.0 - is_ign_f, jnp.float32)
        acc_c = acc_c + jnp.where(lanes == 0, jnp.full(
            (L,), is_ign_f, jnp.float32), 0.0)
    st_v[0, :] = acc_g
    st_v[1, :] = acc_h
    st_v[2, :] = acc_c

    base_s = wid * ROWS_W
    accs0 = tuple(jnp.zeros((L,), jnp.float32) for _ in range(TILE_R))

    def rb_body(rb, accs):
        r0 = base_s + rb * TILE_R
        _fire_half(out_hbm, sbuf_v, r0, 0, 0, sem_a)
        _fire_half(out_hbm, sbuf_v, r0, HALF, 8, sem_b)

        def pair_body(j, a):
            _drain_half(out_hbm, sbuf_v, 0, sem_a)
            a = _sum_half(sbuf_v, 0, a)
            _fire_half(out_hbm, sbuf_v, r0, (2 * j + 2) * HALF, 0, sem_a)
            _drain_half(out_hbm, sbuf_v, 8, sem_b)
            a = _sum_half(sbuf_v, 8, a)
            _fire_half(out_hbm, sbuf_v, r0, (2 * j + 3) * HALF, 8, sem_b)
            return a

        accs = lax.fori_loop(0, N_HALF // 2 - 1, pair_body, accs)
        _drain_half(out_hbm, sbuf_v, 0, sem_a)
        accs = _sum_half(sbuf_v, 0, accs)
        _drain_half(out_hbm, sbuf_v, 8, sem_b)
        accs = _sum_half(sbuf_v, 8, accs)
        pltpu.sync_copy(
            out_hbm.at[pl.ds(r0, TILE_R), pl.ds(TAIL_S0, TAIL_SW)], stail_v)

        def tail_body(i, a):
            c = i * (2 * L)
            a = list(a)
            for cc in range(2):
                for r in range(8):
                    a[r] = a[r] + stail_v[r, pl.ds(c + cc * L, L)]
            return tuple(a)

        return lax.fori_loop(0, TAIL_SW // (2 * L), tail_body, accs)

    accs = lax.fori_loop(0, N_RB_S, rb_body, accs0)
    acc_s = accs[0]
    for r in range(1, TILE_R):
        acc_s = acc_s + accs[r]
    st_v[3, :] = acc_s
    pltpu.sync_copy(st_v, part_hbm.at[wid])


_sc_gather = functools.partial(
    pl.kernel,
    out_type=jax.ShapeDtypeStruct((NW, 4, L), jnp.float32),
    mesh=plsc.VectorSubcoreMesh(core_axis_name="c", subcore_axis_name="s"),
    scratch_types=[
        pltpu.VMEM((B_PER_W,), jnp.int32),
        pltpu.VMEM((B_PER_W, TILE_R, TILE_C), jnp.float32),
        pltpu.VMEM((N_RB, TILE_R, TAIL_W), jnp.float32),
        pltpu.VMEM((N_RB, TILE_R, TILE_C), jnp.float32),
        pltpu.VMEM((16, TILE_R, PANEL), jnp.float32),
        pltpu.VMEM((TILE_R, TAIL_SW), jnp.float32),
        pltpu.VMEM((4, L), jnp.float32),
        pltpu.SemaphoreType.DMA,
        pltpu.SemaphoreType.DMA,
        pltpu.SemaphoreType.DMA,
    ],
)(_sc_gather_body)


_SUM_BR = 64


def _tc_sum_body(x_ref, o_ref):
    @pl.when(pl.program_id(0) == 0)
    def _init():
        o_ref[...] = jnp.zeros_like(o_ref)

    o_ref[...] += jnp.sum(x_ref[...])[None, None]


def kernel(output, target):
    parts = _sc_gather(output, target.astype(jnp.int32))
    g = jnp.sum(parts[:, 0, :])
    h = jnp.sum(parts[:, 1, :])
    cnt = jnp.sum(parts[:, 2, :])

    if R_SC < BATCH:
        tc_total = pl.pallas_call(
            _tc_sum_body,
            grid=((BATCH - R_SC) // _SUM_BR,),
            in_specs=[pl.BlockSpec((_SUM_BR, VOCAB),
                                   lambda i: (R_SC // _SUM_BR + i, 0))],
            out_specs=pl.BlockSpec((1, 1), lambda i: (0, 0)),
            out_shape=jax.ShapeDtypeStruct((1, 1), jnp.float32),
        )(output)[0, 0]
    else:
        tc_total = jnp.float32(0.0)
    total = tc_total + jnp.sum(parts[:, 3, :])

    s = np.float32(LABEL_SMOOTHING / (VOCAB - 2))
    conf = np.float32(CONFIDENCE)
    plogp = (BATCH * conf * np.float32(np.log(CONFIDENCE))
             + s * np.float32(np.log(s)) * (BATCH * (VOCAB - 2) + cnt))
    return plogp - (s * total + (conf - s) * g - s * h)


# speedup vs baseline: 1.0059x; 1.0059x over previous
import functools

import jax
import jax.numpy as jnp
import numpy as np
from jax import lax
from jax.experimental import pallas as pl
from jax.experimental.pallas import tpu as pltpu
from jax.experimental.pallas import tpu_sc as plsc

LABEL_SMOOTHING = 0.1
VOCAB = 100000
CONFIDENCE = 1.0 - LABEL_SMOOTHING
BATCH = 1024
IGN = VOCAB - 100

NC, NS, L = 2, 16, 16
NW = NC * NS
B_PER_W = BATCH // NW
IGN_C0 = (IGN // L) * L
IGN_LANE = IGN - IGN_C0


TILE_R, TILE_C = 8, 128
TAIL0 = (VOCAB // TILE_C) * TILE_C
TAIL_W = VOCAB - TAIL0
LAST_FULL = TAIL0 - TILE_C
IGN_T0 = (IGN // TILE_C) * TILE_C
N_RB = B_PER_W // TILE_R


R_SC = 1024
ROWS_W = R_SC // NW
N_RB_S = ROWS_W // TILE_R
PANEL = 1024
SLOTS_H = 4
HALF = SLOTS_H * PANEL
N_HALF = 24
TAIL_S0 = N_HALF * HALF
TAIL_SW = VOCAB - TAIL_S0


def _sum_half(sbuf_v, s0, accs):
    for b in range(SLOTS_H):
        def body(i, a, _b=b):
            c = i * (4 * L)
            a = list(a)
            for cc in range(4):
                for r in range(8):
                    a[r] = a[r] + sbuf_v[s0 + _b, r, pl.ds(c + cc * L, L)]
            return tuple(a)
        accs = lax.fori_loop(0, PANEL // (4 * L), body, accs)
    return accs


def _fire_half(out_hbm, sbuf_v, r0, col0, s0, sem):
    for b in range(SLOTS_H):
        pltpu.async_copy(
            out_hbm.at[pl.ds(r0, TILE_R),
                       pl.ds(pl.multiple_of(col0 + b * PANEL, 128), PANEL)],
            sbuf_v.at[s0 + b], sem)


def _drain_half(out_hbm, sbuf_v, s0, sem):
    for b in range(SLOTS_H):
        pltpu.make_async_copy(
            out_hbm.at[pl.ds(0, TILE_R), pl.ds(0, PANEL)],
            sbuf_v.at[s0 + b], sem).wait()


def _sc_gather_body(out_hbm, tgt_hbm, part_hbm, tgt_v, gbuf_v, tbuf_v,
                    ibuf_v, sbuf_v, stail_v, st_v, sem, sem_a, sem_b):
    wid = lax.axis_index("s") * NC + lax.axis_index("c")
    base = wid * B_PER_W
    pltpu.sync_copy(tgt_hbm.at[pl.ds(base, B_PER_W)], tgt_v)
    tvecs = [tgt_v[pl.ds(k * L, L)] for k in range(B_PER_W // L)]
    ts = [tvecs[i // L][i % L] for i in range(B_PER_W)]
    copies = []
    for k in range(N_RB):
        r0 = base + k * TILE_R
        copies.append(pltpu.async_copy(
            out_hbm.at[pl.ds(r0, TILE_R), pl.ds(TAIL0, TAIL_W)],
            tbuf_v.at[k], sem))
        copies.append(pltpu.async_copy(
            out_hbm.at[pl.ds(r0, TILE_R), pl.ds(IGN_T0, TILE_C)],
            ibuf_v.at[k], sem))
    for i in range(B_PER_W):
        t = ts[i]
        c0 = pl.multiple_of(
            jnp.minimum((t >> 7) << 7, LAST_FULL), TILE_C)
        r0 = base + (i // TILE_R) * TILE_R
        copies.append(pltpu.async_copy(
            out_hbm.at[pl.ds(r0, TILE_R), pl.ds(c0, TILE_C)],
            gbuf_v.at[i], sem))
    for c in copies:
        c.wait()
    lanes = lax.iota(jnp.int32, L)
    acc_g = jnp.zeros((L,), jnp.float32)
    acc_h = jnp.zeros((L,), jnp.float32)
    acc_c = jnp.zeros((L,), jnp.float32)
    for i in range(B_PER_W):
        t = ts[i]
        below_f = jnp.where(t < TAIL0, 1.0, 0.0)
        is_ign_f = jnp.where(t == IGN, 1.0, 0.0)
        off = t - jnp.minimum((t >> 7) << 7, LAST_FULL)
        sub = jnp.minimum((off >> 4) << 4, TILE_C - L)
        vm = gbuf_v[i, i % TILE_R, pl.ds(sub, L)]
        lane_m = jnp.full((L,), off - sub, jnp.int32)
        acc_g = acc_g + jnp.where(lanes == lane_m, vm, 0.0) * jnp.full(
            (L,), below_f, jnp.float32)
        offt = jnp.maximum(t - TAIL0, 0)
        subt = jnp.minimum((offt >> 4) << 4, TAIL_W - L)
        vt = tbuf_v[i // TILE_R, i % TILE_R, pl.ds(subt, L)]
        lane_t = jnp.full((L,), offt - subt, jnp.int32)
        acc_g = acc_g + jnp.where(lanes == lane_t, vt, 0.0) * jnp.full(
            (L,), 1.0 - below_f, jnp.float32)
        ign_sub = ((IGN - IGN_T0) // L) * L
        ign_lane = (IGN - IGN_T0) - ign_sub
        vi = ibuf_v[i // TILE_R, i % TILE_R, pl.ds(ign_sub, L)]
        acc_h = acc_h + jnp.where(lanes == ign_lane, vi, 0.0) * jnp.full(
            (L,), 1.0 - is_ign_f, jnp.float32)
        acc_c = acc_c + jnp.where(lanes == 0, jnp.full(
            (L,), is_ign_f, jnp.float32), 0.0)
    st_v[0, :] = acc_g
    st_v[1, :] = acc_h
    st_v[2, :] = acc_c

    base_s = wid * ROWS_W
    accs0 = tuple(jnp.zeros((L,), jnp.float32) for _ in range(TILE_R))

    def rb_body(rb, accs):
        r0 = base_s + rb * TILE_R
        _fire_half(out_hbm, sbuf_v, r0, 0, 0, sem_a)
        _fire_half(out_hbm, sbuf_v, r0, HALF, SLOTS_H, sem_b)

        def pair_body(j, a):
            _drain_half(out_hbm, sbuf_v, 0, sem_a)
            a = _sum_half(sbuf_v, 0, a)
            _fire_half(out_hbm, sbuf_v, r0, (2 * j + 2) * HALF, 0, sem_a)
            _drain_half(out_hbm, sbuf_v, SLOTS_H, sem_b)
            a = _sum_half(sbuf_v, SLOTS_H, a)
            _fire_half(out_hbm, sbuf_v, r0, (2 * j + 3) * HALF, SLOTS_H, sem_b)
            return a

        accs = lax.fori_loop(0, N_HALF // 2 - 1, pair_body, accs)
        _drain_half(out_hbm, sbuf_v, 0, sem_a)
        accs = _sum_half(sbuf_v, 0, accs)
        _drain_half(out_hbm, sbuf_v, SLOTS_H, sem_b)
        accs = _sum_half(sbuf_v, SLOTS_H, accs)
        pltpu.sync_copy(
            out_hbm.at[pl.ds(r0, TILE_R), pl.ds(TAIL_S0, TAIL_SW)], stail_v)

        def tail_body(i, a):
            c = i * (2 * L)
            a = list(a)
            for cc in range(2):
                for r in range(8):
                    a[r] = a[r] + stail_v[r, pl.ds(c + cc * L, L)]
            return tuple(a)

        return lax.fori_loop(0, TAIL_SW // (2 * L), tail_body, accs)

    accs = lax.fori_loop(0, N_RB_S, rb_body, accs0)
    acc_s = accs[0]
    for r in range(1, TILE_R):
        acc_s = acc_s + accs[r]
    st_v[3, :] = acc_s
    pltpu.sync_copy(st_v, part_hbm.at[wid])


_sc_gather = functools.partial(
    pl.kernel,
    out_type=jax.ShapeDtypeStruct((NW, 4, L), jnp.float32),
    mesh=plsc.VectorSubcoreMesh(core_axis_name="c", subcore_axis_name="s"),
    scratch_types=[
        pltpu.VMEM((B_PER_W,), jnp.int32),
        pltpu.VMEM((B_PER_W, TILE_R, TILE_C), jnp.float32),
        pltpu.VMEM((N_RB, TILE_R, TAIL_W), jnp.float32),
        pltpu.VMEM((N_RB, TILE_R, TILE_C), jnp.float32),
        pltpu.VMEM((2 * SLOTS_H, TILE_R, PANEL), jnp.float32),
        pltpu.VMEM((TILE_R, TAIL_SW), jnp.float32),
        pltpu.VMEM((4, L), jnp.float32),
        pltpu.SemaphoreType.DMA,
        pltpu.SemaphoreType.DMA,
        pltpu.SemaphoreType.DMA,
    ],
)(_sc_gather_body)


_SUM_BR = 64


def _tc_sum_body(x_ref, o_ref):
    @pl.when(pl.program_id(0) == 0)
    def _init():
        o_ref[...] = jnp.zeros_like(o_ref)

    o_ref[...] += jnp.sum(x_ref[...])[None, None]


def kernel(output, target):
    parts = _sc_gather(output, target.astype(jnp.int32))
    g = jnp.sum(parts[:, 0, :])
    h = jnp.sum(parts[:, 1, :])
    cnt = jnp.sum(parts[:, 2, :])

    if R_SC < BATCH:
        tc_total = pl.pallas_call(
            _tc_sum_body,
            grid=((BATCH - R_SC) // _SUM_BR,),
            in_specs=[pl.BlockSpec((_SUM_BR, VOCAB),
                                   lambda i: (R_SC // _SUM_BR + i, 0))],
            out_specs=pl.BlockSpec((1, 1), lambda i: (0, 0)),
            out_shape=jax.ShapeDtypeStruct((1, 1), jnp.float32),
        )(output)[0, 0]
    else:
        tc_total = jnp.float32(0.0)
    total = tc_total + jnp.sum(parts[:, 3, :])

    s = np.float32(LABEL_SMOOTHING / (VOCAB - 2))
    conf = np.float32(CONFIDENCE)
    plogp = (BATCH * conf * np.float32(np.log(CONFIDENCE))
             + s * np.float32(np.log(s)) * (BATCH * (VOCAB - 2) + cnt))
    return plogp - (s * total + (conf - s) * g - s * h)


# speedup vs baseline: 1.1028x; 1.0963x over previous
import functools

import jax
import jax.numpy as jnp
import numpy as np
from jax import lax
from jax.experimental import pallas as pl
from jax.experimental.pallas import tpu as pltpu
from jax.experimental.pallas import tpu_sc as plsc

LABEL_SMOOTHING = 0.1
VOCAB = 100000
CONFIDENCE = 1.0 - LABEL_SMOOTHING
BATCH = 1024
IGN = VOCAB - 100

NC, NS, L = 2, 16, 16
NW = NC * NS
B_PER_W = BATCH // NW
IGN_C0 = (IGN // L) * L
IGN_LANE = IGN - IGN_C0


TILE_R, TILE_C = 8, 128
TAIL0 = (VOCAB // TILE_C) * TILE_C
TAIL_W = VOCAB - TAIL0
LAST_FULL = TAIL0 - TILE_C
IGN_T0 = (IGN // TILE_C) * TILE_C
N_RB = B_PER_W // TILE_R


R_SC = 0
ROWS_W = R_SC // NW
N_RB_S = ROWS_W // TILE_R
PANEL = 1024
SLOTS_H = 4
HALF = SLOTS_H * PANEL
N_HALF = 24
TAIL_S0 = N_HALF * HALF
TAIL_SW = VOCAB - TAIL_S0


def _sum_half(sbuf_v, s0, accs):
    for b in range(SLOTS_H):
        def body(i, a, _b=b):
            c = i * (4 * L)
            a = list(a)
            for cc in range(4):
                for r in range(8):
                    a[r] = a[r] + sbuf_v[s0 + _b, r, pl.ds(c + cc * L, L)]
            return tuple(a)
        accs = lax.fori_loop(0, PANEL // (4 * L), body, accs)
    return accs


def _fire_half(out_hbm, sbuf_v, r0, col0, s0, sem):
    for b in range(SLOTS_H):
        pltpu.async_copy(
            out_hbm.at[pl.ds(r0, TILE_R),
                       pl.ds(pl.multiple_of(col0 + b * PANEL, 128), PANEL)],
            sbuf_v.at[s0 + b], sem)


def _drain_half(out_hbm, sbuf_v, s0, sem):
    for b in range(SLOTS_H):
        pltpu.make_async_copy(
            out_hbm.at[pl.ds(0, TILE_R), pl.ds(0, PANEL)],
            sbuf_v.at[s0 + b], sem).wait()


def _sc_gather_body(out_hbm, tgt_hbm, part_hbm, tgt_v, gbuf_v, tbuf_v,
                    ibuf_v, sbuf_v, stail_v, st_v, sem, sem_a, sem_b):
    wid = lax.axis_index("s") * NC + lax.axis_index("c")
    base = wid * B_PER_W
    pltpu.sync_copy(tgt_hbm.at[pl.ds(base, B_PER_W)], tgt_v)
    tvecs = [tgt_v[pl.ds(k * L, L)] for k in range(B_PER_W // L)]
    ts = [tvecs[i // L][i % L] for i in range(B_PER_W)]
    copies = []
    for k in range(N_RB):
        r0 = base + k * TILE_R
        copies.append(pltpu.async_copy(
            out_hbm.at[pl.ds(r0, TILE_R), pl.ds(TAIL0, TAIL_W)],
            tbuf_v.at[k], sem))
        copies.append(pltpu.async_copy(
            out_hbm.at[pl.ds(r0, TILE_R), pl.ds(IGN_T0, TILE_C)],
            ibuf_v.at[k], sem))
    for i in range(B_PER_W):
        t = ts[i]
        c0 = pl.multiple_of(
            jnp.minimum((t >> 7) << 7, LAST_FULL), TILE_C)
        r0 = base + (i // TILE_R) * TILE_R
        copies.append(pltpu.async_copy(
            out_hbm.at[pl.ds(r0, TILE_R), pl.ds(c0, TILE_C)],
            gbuf_v.at[i], sem))
    for c in copies:
        c.wait()
    lanes = lax.iota(jnp.int32, L)
    acc_g = jnp.zeros((L,), jnp.float32)
    acc_h = jnp.zeros((L,), jnp.float32)
    acc_c = jnp.zeros((L,), jnp.float32)
    for i in range(B_PER_W):
        t = ts[i]
        below_f = jnp.where(t < TAIL0, 1.0, 0.0)
        is_ign_f = jnp.where(t == IGN, 1.0, 0.0)
        off = t - jnp.minimum((t >> 7) << 7, LAST_FULL)
        sub = jnp.minimum((off >> 4) << 4, TILE_C - L)
        vm = gbuf_v[i, i % TILE_R, pl.ds(sub, L)]
        lane_m = jnp.full((L,), off - sub, jnp.int32)
        acc_g = acc_g + jnp.where(lanes == lane_m, vm, 0.0) * jnp.full(
            (L,), below_f, jnp.float32)
        offt = jnp.maximum(t - TAIL0, 0)
        subt = jnp.minimum((offt >> 4) << 4, TAIL_W - L)
        vt = tbuf_v[i // TILE_R, i % TILE_R, pl.ds(subt, L)]
        lane_t = jnp.full((L,), offt - subt, jnp.int32)
        acc_g = acc_g + jnp.where(lanes == lane_t, vt, 0.0) * jnp.full(
            (L,), 1.0 - below_f, jnp.float32)
        ign_sub = ((IGN - IGN_T0) // L) * L
        ign_lane = (IGN - IGN_T0) - ign_sub
        vi = ibuf_v[i // TILE_R, i % TILE_R, pl.ds(ign_sub, L)]
        acc_h = acc_h + jnp.where(lanes == ign_lane, vi, 0.0) * jnp.full(
            (L,), 1.0 - is_ign_f, jnp.float32)
        acc_c = acc_c + jnp.where(lanes == 0, jnp.full(
            (L,), is_ign_f, jnp.float32), 0.0)
    st_v[0, :] = acc_g
    st_v[1, :] = acc_h
    st_v[2, :] = acc_c

    base_s = wid * ROWS_W
    accs0 = tuple(jnp.zeros((L,), jnp.float32) for _ in range(TILE_R))

    def rb_body(rb, accs):
        r0 = base_s + rb * TILE_R
        _fire_half(out_hbm, sbuf_v, r0, 0, 0, sem_a)
        _fire_half(out_hbm, sbuf_v, r0, HALF, SLOTS_H, sem_b)

        def pair_body(j, a):
            _drain_half(out_hbm, sbuf_v, 0, sem_a)
            a = _sum_half(sbuf_v, 0, a)
            _fire_half(out_hbm, sbuf_v, r0, (2 * j + 2) * HALF, 0, sem_a)
            _drain_half(out_hbm, sbuf_v, SLOTS_H, sem_b)
            a = _sum_half(sbuf_v, SLOTS_H, a)
            _fire_half(out_hbm, sbuf_v, r0, (2 * j + 3) * HALF, SLOTS_H, sem_b)
            return a

        accs = lax.fori_loop(0, N_HALF // 2 - 1, pair_body, accs)
        _drain_half(out_hbm, sbuf_v, 0, sem_a)
        accs = _sum_half(sbuf_v, 0, accs)
        _drain_half(out_hbm, sbuf_v, SLOTS_H, sem_b)
        accs = _sum_half(sbuf_v, SLOTS_H, accs)
        pltpu.sync_copy(
            out_hbm.at[pl.ds(r0, TILE_R), pl.ds(TAIL_S0, TAIL_SW)], stail_v)

        def tail_body(i, a):
            c = i * (2 * L)
            a = list(a)
            for cc in range(2):
                for r in range(8):
                    a[r] = a[r] + stail_v[r, pl.ds(c + cc * L, L)]
            return tuple(a)

        return lax.fori_loop(0, TAIL_SW // (2 * L), tail_body, accs)

    accs = lax.fori_loop(0, N_RB_S, rb_body, accs0)
    acc_s = accs[0]
    for r in range(1, TILE_R):
        acc_s = acc_s + accs[r]
    st_v[3, :] = acc_s
    pltpu.sync_copy(st_v, part_hbm.at[wid])


_sc_gather = functools.partial(
    pl.kernel,
    out_type=jax.ShapeDtypeStruct((NW, 4, L), jnp.float32),
    mesh=plsc.VectorSubcoreMesh(core_axis_name="c", subcore_axis_name="s"),
    scratch_types=[
        pltpu.VMEM((B_PER_W,), jnp.int32),
        pltpu.VMEM((B_PER_W, TILE_R, TILE_C), jnp.float32),
        pltpu.VMEM((N_RB, TILE_R, TAIL_W), jnp.float32),
        pltpu.VMEM((N_RB, TILE_R, TILE_C), jnp.float32),
        pltpu.VMEM((2 * SLOTS_H, TILE_R, PANEL), jnp.float32),
        pltpu.VMEM((TILE_R, TAIL_SW), jnp.float32),
        pltpu.VMEM((4, L), jnp.float32),
        pltpu.SemaphoreType.DMA,
        pltpu.SemaphoreType.DMA,
        pltpu.SemaphoreType.DMA,
    ],
)(_sc_gather_body)


_SUM_BR = 64


def _tc_sum_body(x_ref, o_ref):
    @pl.when(pl.program_id(0) == 0)
    def _init():
        o_ref[...] = jnp.zeros_like(o_ref)

    o_ref[...] += jnp.sum(x_ref[...])[None, None]


def kernel(output, target):
    parts = _sc_gather(output, target.astype(jnp.int32))
    g = jnp.sum(parts[:, 0, :])
    h = jnp.sum(parts[:, 1, :])
    cnt = jnp.sum(parts[:, 2, :])

    if R_SC < BATCH:
        tc_total = pl.pallas_call(
            _tc_sum_body,
            grid=((BATCH - R_SC) // _SUM_BR,),
            in_specs=[pl.BlockSpec((_SUM_BR, VOCAB),
                                   lambda i: (R_SC // _SUM_BR + i, 0))],
            out_specs=pl.BlockSpec((1, 1), lambda i: (0, 0)),
            out_shape=jax.ShapeDtypeStruct((1, 1), jnp.float32),
        )(output)[0, 0]
    else:
        tc_total = jnp.float32(0.0)
    total = tc_total + jnp.sum(parts[:, 3, :])

    s = np.float32(LABEL_SMOOTHING / (VOCAB - 2))
    conf = np.float32(CONFIDENCE)
    plogp = (BATCH * conf * np.float32(np.log(CONFIDENCE))
             + s * np.float32(np.log(s)) * (BATCH * (VOCAB - 2) + cnt))
    return plogp - (s * total + (conf - s) * g - s * h)
